# Initial kernel scaffold; baseline (speedup 1.0000x reference)
import jax
import jax.numpy as jnp
from jax.experimental import pallas as pl


def kernel(x, pos, batch, params):
    raise NotImplementedError("write your pallas kernel here")


# speedup vs baseline: 14.8131x; 14.8131x over previous
import functools

import jax
import jax.numpy as jnp
from jax import lax
from jax.experimental import pallas as pl
from jax.experimental.pallas import tpu as pltpu


def _fps_body(planes_ref, pos_out_ref, idx_out_ref):
    C = planes_ref.shape[2]
    n = idx_out_ref.shape[0]
    px = planes_ref[0]
    py = planes_ref[1]
    pz = planes_ref[2]
    row = lax.broadcasted_iota(jnp.int32, (8, C), 0)
    col = lax.broadcasted_iota(jnp.int32, (8, C), 1)
    flat = row * C + col
    p0x = px[0:1, 0:1]
    p0y = py[0:1, 0:1]
    p0z = pz[0:1, 0:1]
    pos_out_ref[0:1, 0:1] = p0x
    pos_out_ref[0:1, 1:2] = p0y
    pos_out_ref[0:1, 2:3] = p0z
    idx_out_ref[0:1, :] = jnp.zeros((1, 1), jnp.int32)
    d0 = (px - p0x) ** 2 + (py - p0y) ** 2 + (pz - p0z) ** 2
    big = jnp.int32(2 ** 30)

    def step(t, d):
        m = jnp.max(d)
        fidx = jnp.min(jnp.where(d == m, flat, big))
        mask = flat == fidx
        nx = jnp.sum(jnp.where(mask, px, 0.0))
        ny = jnp.sum(jnp.where(mask, py, 0.0))
        nz = jnp.sum(jnp.where(mask, pz, 0.0))
        pos_out_ref[pl.ds(t, 1), 0:1] = nx.reshape(1, 1)
        pos_out_ref[pl.ds(t, 1), 1:2] = ny.reshape(1, 1)
        pos_out_ref[pl.ds(t, 1), 2:3] = nz.reshape(1, 1)
        idx_out_ref[pl.ds(t, 1), :] = fidx.reshape(1, 1)
        nd = (px - nx) ** 2 + (py - ny) ** 2 + (pz - nz) ** 2
        return jnp.minimum(d, nd)

    lax.fori_loop(1, n, step, d0)


def _fps(pos_rows, n_samples):
    N = pos_rows.shape[0]
    planes = pos_rows.T.reshape(3, 8, N // 8)
    pos_s, idx = pl.pallas_call(
        _fps_body,
        out_shape=[
            jax.ShapeDtypeStruct((n_samples, 3), jnp.float32),
            jax.ShapeDtypeStruct((n_samples, 1), jnp.int32),
        ],
    )(planes)
    return pos_s, idx.reshape(n_samples)


def _gather_mlp_body(*refs, n_w, has_pos):
    idx_ref, src_ref = refs[0], refs[1]
    pos_ref = refs[2] if has_pos else None
    wrefs = refs[(3 if has_pos else 2):-2]
    out_ref, gat = refs[-2], refs[-1]
    n_out = gat.shape[0]

    def body(t, carry):
        gat[pl.ds(t, 1), :] = src_ref[pl.ds(idx_ref[t], 1), :]
        return carry

    lax.fori_loop(0, n_out, body, 0)
    g = gat[...]
    if has_pos:
        wf, wp, b1 = wrefs[0][...], wrefs[1][...], wrefs[2][...]
        h = jnp.dot(g, wf, preferred_element_type=jnp.float32)
        h = h + jnp.dot(pos_ref[...], wp, preferred_element_type=jnp.float32)
        h = jnp.maximum(h + b1, 0.0)
        wrefs = wrefs[3:]
    else:
        h = g
    for i in range(len(wrefs) // 2):
        w, b = wrefs[2 * i][...], wrefs[2 * i + 1][...]
        h = jnp.maximum(jnp.dot(h, w, preferred_element_type=jnp.float32) + b, 0.0)
    out_ref[...] = h


def _gather_mlp(idx, src, pos_s, weights, out_dim):
    n = idx.shape[0]
    args = [idx, src]
    in_specs = [pl.BlockSpec(memory_space=pltpu.SMEM), pl.BlockSpec()]
    if pos_s is not None:
        args.append(pos_s)
        in_specs.append(pl.BlockSpec())
    for w in weights:
        args.append(w)
        in_specs.append(pl.BlockSpec())
    body = functools.partial(_gather_mlp_body, n_w=len(weights),
                             has_pos=pos_s is not None)
    return pl.pallas_call(
        body,
        out_shape=jax.ShapeDtypeStruct((n, out_dim), jnp.float32),
        in_specs=in_specs,
        scratch_shapes=[pltpu.VMEM((n, src.shape[1]), jnp.float32)],
    )(*args)


def _interp_mlp_body(*refs, final_head):
    qpos_ref, spos_t_ref, feats_ref, skip_ref = refs[:4]
    wrefs = refs[4:-1]
    out_ref = refs[-1]
    q = qpos_ref[...]
    st = spos_t_ref[...]
    S = st.shape[1]
    QB = q.shape[0]
    qs = jnp.dot(q, st, preferred_element_type=jnp.float32)
    q2 = jnp.sum(q * q, axis=1, keepdims=True)
    s2 = jnp.sum(st * st, axis=0, keepdims=True)
    d2 = jnp.maximum(q2 + s2 - 2.0 * qs, 0.0)
    jcol = lax.broadcasted_iota(jnp.int32, (QB, S), 1)
    inf = jnp.float32(jnp.inf)
    eps = jnp.float32(1e-16)

    m1 = jnp.min(d2, axis=1, keepdims=True)
    i1 = jnp.min(jnp.where(d2 == m1, jcol, S), axis=1, keepdims=True)
    d2a = jnp.where(jcol == i1, inf, d2)
    m2 = jnp.min(d2a, axis=1, keepdims=True)
    i2 = jnp.min(jnp.where(d2a == m2, jcol, S), axis=1, keepdims=True)
    d2b = jnp.where(jcol == i2, inf, d2a)
    m3 = jnp.min(d2b, axis=1, keepdims=True)
    i3 = jnp.min(jnp.where(d2b == m3, jcol, S), axis=1, keepdims=True)

    w1 = 1.0 / jnp.maximum(m1, eps)
    w2 = 1.0 / jnp.maximum(m2, eps)
    w3 = 1.0 / jnp.maximum(m3, eps)
    wmat = (jnp.where(jcol == i1, w1, 0.0)
            + jnp.where(jcol == i2, w2, 0.0)
            + jnp.where(jcol == i3, w3, 0.0))
    sumw = w1 + w2 + w3
    interp = jnp.dot(wmat, feats_ref[...], preferred_element_type=jnp.float32)
    interp = interp * (1.0 / sumw)

    wa, wb, b1 = wrefs[0][...], wrefs[1][...], wrefs[2][...]
    h = jnp.dot(interp, wa, preferred_element_type=jnp.float32)
    h = h + jnp.dot(skip_ref[...], wb, preferred_element_type=jnp.float32)
    h = jnp.maximum(h + b1, 0.0)
    wrefs = wrefs[3:]
    n_rest = len(wrefs) // 2
    for i in range(n_rest):
        w, b = wrefs[2 * i][...], wrefs[2 * i + 1][...]
        h = jnp.dot(h, w, preferred_element_type=jnp.float32) + b
        if i < n_rest - 1 or not final_head:
            h = jnp.maximum(h, 0.0)
    out_ref[...] = h


def _interp_mlp(qpos, spos_t, feats, skip, weights, out_dim, qblock,
                final_head=False):
    NQ = qpos.shape[0]
    grid = NQ // qblock
    S = spos_t.shape[1]
    in_specs = [
        pl.BlockSpec((qblock, 3), lambda i: (i, 0)),
        pl.BlockSpec((3, S), lambda i: (0, 0)),
        pl.BlockSpec(feats.shape, lambda i: (0, 0)),
        pl.BlockSpec((qblock, skip.shape[1]), lambda i: (i, 0)),
    ]
    args = [qpos, spos_t, feats, skip]
    for w in weights:
        args.append(w)
        in_specs.append(pl.BlockSpec(w.shape, lambda i: (0, 0)))
    body = functools.partial(_interp_mlp_body, final_head=final_head)
    return pl.pallas_call(
        body,
        grid=(grid,),
        in_specs=in_specs,
        out_specs=pl.BlockSpec((qblock, out_dim), lambda i: (i, 0)),
        out_shape=jax.ShapeDtypeStruct((NQ, out_dim), jnp.float32),
    )(*args)


def _split_first(mlp_params, split):
    w1t = mlp_params[0][0].T
    out = [w1t[:split], w1t[split:], mlp_params[0][1].reshape(1, -1)]
    for (W, b) in mlp_params[1:]:
        out.append(W.T)
        out.append(b.reshape(1, -1))
    return out


def kernel(x, pos, batch, params):
    N = pos.shape[0]

    l1_pos, l1_idx = _fps(pos, N // 4)
    l2_pos, l2_idx = _fps(l1_pos, N // 16)
    l3_pos, l3_idx = _fps(l2_pos, N // 64)

    sa1_w = []
    for (W, b) in params['sa1']:
        sa1_w.append(W.T)
        sa1_w.append(b.reshape(1, -1))
    l1_x = _gather_mlp(l1_idx, x, None, sa1_w, 128)

    l2_x = _gather_mlp(l2_idx, l1_x, l2_pos, _split_first(params['sa2'], 128), 256)
    l3_f = _gather_mlp(l3_idx, l2_x, l3_pos, _split_first(params['sa3'], 256), 1024)

    l2_fp = _interp_mlp(l2_pos, l3_pos.T, l3_f, l2_x,
                        _split_first(params['fp3'], 1024), 256, 512)
    l1_fp = _interp_mlp(l1_pos, l2_pos.T, l2_fp, l1_x,
                        _split_first(params['fp2'], 256), 128, 256)

    fp1_w = _split_first(params['fp1'], 128)
    for (W, b) in params['head']:
        fp1_w.append(W.T)
        fp1_w.append(b.reshape(1, -1))
    alpha = _interp_mlp(pos, l1_pos.T, l1_fp, x, fp1_w, 1, 128,
                        final_head=True)

    dense = alpha.reshape(1, 1, N)
    mean = jnp.maximum(dense, 0.0) + jnp.log1p(jnp.exp(-jnp.abs(dense)))
    std = jnp.ones_like(mean) * 0.01
    return (mean, std)
